# baseline (speedup 1.0000x reference)
import functools

import jax
import jax.numpy as jnp
from jax import lax
from jax.experimental import pallas as pl
from jax.experimental.pallas import tpu as pltpu
from jax.experimental.pallas import tpu_sc as plsc

Q_MAX = 127.0
QUANTILE = 0.99
GAMMA_MIN = 0.1
GAMMA_MAX = 10.0

NC = 2
NS = 16
L = 16
NW = NC * NS
NBUCKET = 65536
CHUNK = 8192


RB = 8


def _sc_hist(x_bits, b16):
    a_dim, m_dim, w_dim = x_bits.shape
    rb_per_a = m_dim // RB
    n_chunks = a_dim * rb_per_a // NW
    assert m_dim % RB == 0 and (a_dim * rb_per_a) % NW == 0
    assert n_chunks % 2 == 0
    nvec = RB * w_dim // L
    pass2 = b16 is not None
    mesh = plsc.VectorSubcoreMesh(core_axis_name="c", subcore_axis_name="s")

    hist_shape = (512, 128) if pass2 else (2, 256, 128)
    scratch = [
        pltpu.VMEM(hist_shape, jnp.int32),
        pltpu.VMEM((RB, w_dim), jnp.float32),
        pltpu.VMEM((RB, w_dim), jnp.float32),
        pltpu.SemaphoreType.DMA,
        pltpu.SemaphoreType.DMA,
    ]
    if pass2:
        scratch.append(pltpu.VMEM((L,), jnp.int32))

    def body(*refs):
        if pass2:
            x_hbm, b_hbm, out_hbm, hist, buf0, buf1, sem0, sem1, bvec = refs
        else:
            x_hbm, out_hbm, hist, buf0, buf1, sem0, sem1 = refs
        bufs = (buf0, buf1)
        sems = (sem0, sem1)
        wid = lax.axis_index("s") * NC + lax.axis_index("c")
        base = wid * n_chunks

        rb_shift = rb_per_a.bit_length() - 1
        assert rb_per_a == 1 << rb_shift

        def start_dma(unit, j):
            a = lax.shift_right_logical(unit, rb_shift)
            rb = unit & (rb_per_a - 1)
            pltpu.make_async_copy(
                x_hbm.at[a, pl.ds(rb * RB, RB), :], bufs[j], sems[j]
            ).start()

        zeros = jnp.zeros((L,), jnp.int32)

        if pass2:

            @plsc.parallel_loop(0, 512, unroll=4)
            def _zero(r):
                for c in range(8):
                    hist[r, pl.ds(c * L, L)] = zeros

        else:

            @plsc.parallel_loop(0, 256, unroll=4)
            def _zero(r):
                for h in range(2):
                    for c in range(8):
                        hist[h, r, pl.ds(c * L, L)] = zeros

        if pass2:
            pltpu.sync_copy(b_hbm, bvec)
            bsplat16 = bvec[...] << 16

        ones = jnp.ones((L,), jnp.int32)

        for j in range(2):
            start_dma(base + j, j)

        wpl = w_dim // L
        wpl_shift = wpl.bit_length() - 1
        assert wpl == 1 << wpl_shift

        @pl.loop(0, n_chunks, step=2)
        def _outer(g):
            for j in range(2):
                gg = g + j
                pltpu.make_async_copy(
                    x_hbm.at[0, pl.ds(0, RB), :], bufs[j], sems[j]
                ).wait()

                if pass2:

                    @plsc.parallel_loop(0, nvec, unroll=16)
                    def _inner(i):
                        r = lax.shift_right_logical(i, wpl_shift)
                        c = (i & (wpl - 1)) * L
                        bits = plsc.bitcast(
                            bufs[j][r, pl.ds(c, L)], jnp.int32
                        )
                        hi16 = bits & 0x7FFF0000
                        row = lax.shift_right_logical(bits, 7) & 0x1FF
                        col = bits & 127
                        plsc.addupdate_scatter(
                            hist, [row, col], ones, mask=(hi16 == bsplat16)
                        )

                else:

                    @plsc.parallel_loop(0, nvec // 2, unroll=8)
                    def _inner(i):
                        i2 = i * 2
                        r = lax.shift_right_logical(i2, wpl_shift)
                        c = (i2 & (wpl - 1)) * L
                        for h in range(2):
                            bits = plsc.bitcast(
                                bufs[j][r, pl.ds(c + h * L, L)], jnp.int32
                            )
                            row = lax.shift_right_logical(bits, 23) & 0xFF
                            col = lax.shift_right_logical(bits, 16) & 127
                            plsc.addupdate_scatter(
                                hist.at[h], [row, col], ones
                            )

                @pl.when(gg + 2 < n_chunks)
                def _next():
                    start_dma(base + gg + 2, j)

        pltpu.sync_copy(hist, out_hbm.at[wid])

    kern = pl.kernel(
        body,
        out_type=jax.ShapeDtypeStruct((NW,) + hist_shape, jnp.int32),
        mesh=mesh,
        scratch_types=scratch,
        compiler_params=pltpu.CompilerParams(
            needs_layout_passes=False, use_tc_tiling_on_sc=True
        ),
    )
    if pass2:
        return kern(x_bits, b16)
    return kern(x_bits)


def _cumulative(s):
    rows = s.shape[0]
    c = s
    sh = 1
    while sh < 128:
        c = c + jnp.concatenate(
            [jnp.zeros((rows, sh), jnp.int32), c[:, :-sh]], axis=1
        )
        sh *= 2
    rt = c[:, 127:128]
    e = rt
    sh = 1
    while sh < rows:
        e = e + jnp.concatenate(
            [jnp.zeros((sh, 1), jnp.int32), e[:-sh, :]], axis=0
        )
        sh *= 2
    return (e - rt) + c


def _tc_reduce1(h1, k):
    def body(h_ref, b_ref, r_ref):
        cum = _cumulative(jnp.sum(h_ref[...], axis=(0, 1)))
        mask = cum <= k
        b = jnp.sum(mask.astype(jnp.int32))
        cum_before = jnp.max(jnp.where(mask, cum, 0))
        b_ref[...] = jnp.full((1, L), b, jnp.int32)
        r_ref[...] = jnp.full((1, 1), k - cum_before, jnp.int32)

    return pl.pallas_call(
        body,
        out_shape=(
            jax.ShapeDtypeStruct((1, L), jnp.int32),
            jax.ShapeDtypeStruct((1, 1), jnp.int32),
        ),
    )(h1)


def _tc_reduce2(h2, b2d, r2d, g2d):
    def body(h_ref, b_ref, r_ref, g_ref, s_ref):
        cum = _cumulative(jnp.sum(h_ref[...], axis=0))
        r = r_ref[0, 0]
        low = jnp.sum((cum <= r).astype(jnp.int32))
        qbits = (b_ref[0, 0] << 16) | low
        q = lax.bitcast_convert_type(qbits, jnp.float32)
        gc = jnp.clip(g_ref[0, 0], GAMMA_MIN, GAMMA_MAX)
        s_ref[...] = jnp.full((1, 1), q * gc / Q_MAX, jnp.float32)

    return pl.pallas_call(
        body,
        out_shape=jax.ShapeDtypeStruct((1, 1), jnp.float32),
    )(h2, b2d, r2d, g2d)


def _tc_quantize(x3d, scale):
    a, m, w = x3d.shape
    bm = 512

    def body(s_ref, x_ref, o_ref):
        s = s_ref[0, 0]
        si = 1.0 / s
        q = jnp.clip(jnp.round(x_ref[...] * si), -Q_MAX, Q_MAX)
        o_ref[...] = q * s

    return pl.pallas_call(
        body,
        grid=(a, m // bm),
        in_specs=[
            pl.BlockSpec(memory_space=pltpu.SMEM),
            pl.BlockSpec((1, bm, w), lambda i, j: (i, j, 0)),
        ],
        out_specs=pl.BlockSpec((1, bm, w), lambda i, j: (i, j, 0)),
        out_shape=jax.ShapeDtypeStruct((a, m, w), jnp.float32),
    )(scale, x3d)


def kernel(x, gamma):
    n = x.size
    k = round(QUANTILE * n)
    h1 = _sc_hist(x, None)
    b2d, r2d = _tc_reduce1(h1, k)
    h2 = _sc_hist(x, b2d.reshape(L))
    scale = _tc_reduce2(h2, b2d, r2d, gamma.reshape(1, 1))
    return _tc_quantize(x, scale)


# speedup vs baseline: 1.0230x; 1.0230x over previous
import functools

import jax
import jax.numpy as jnp
from jax import lax
from jax.experimental import pallas as pl
from jax.experimental.pallas import tpu as pltpu
from jax.experimental.pallas import tpu_sc as plsc

Q_MAX = 127.0
QUANTILE = 0.99
GAMMA_MIN = 0.1
GAMMA_MAX = 10.0

NC = 2
NS = 16
L = 16
NW = NC * NS
NBUCKET = 65536
CHUNK = 8192


RB = 8


def _sc_hist(x_bits, b16):
    a_dim, m_dim, w_dim = x_bits.shape
    rb_per_a = m_dim // RB
    n_chunks = a_dim * rb_per_a // NW
    assert m_dim % RB == 0 and (a_dim * rb_per_a) % NW == 0
    assert n_chunks % 2 == 0
    nvec = RB * w_dim // L
    pass2 = b16 is not None
    mesh = plsc.VectorSubcoreMesh(core_axis_name="c", subcore_axis_name="s")

    hist_shape = (512, 128)
    scratch = [
        pltpu.VMEM(hist_shape, jnp.int32),
        pltpu.VMEM((RB, w_dim), jnp.float32),
        pltpu.VMEM((RB, w_dim), jnp.float32),
        pltpu.SemaphoreType.DMA,
        pltpu.SemaphoreType.DMA,
    ]
    if pass2:
        scratch.append(pltpu.VMEM((L,), jnp.int32))

    def body(*refs):
        if pass2:
            x_hbm, b_hbm, out_hbm, hist, buf0, buf1, sem0, sem1, bvec = refs
        else:
            x_hbm, out_hbm, hist, buf0, buf1, sem0, sem1 = refs
        bufs = (buf0, buf1)
        sems = (sem0, sem1)
        wid = lax.axis_index("s") * NC + lax.axis_index("c")
        base = wid * n_chunks

        rb_shift = rb_per_a.bit_length() - 1
        assert rb_per_a == 1 << rb_shift

        def start_dma(unit, j):
            a = lax.shift_right_logical(unit, rb_shift)
            rb = unit & (rb_per_a - 1)
            pltpu.make_async_copy(
                x_hbm.at[a, pl.ds(rb * RB, RB), :], bufs[j], sems[j]
            ).start()

        zeros = jnp.zeros((L,), jnp.int32)

        @plsc.parallel_loop(0, 512, unroll=4)
        def _zero(r):
            for c in range(8):
                hist[r, pl.ds(c * L, L)] = zeros

        if pass2:
            pltpu.sync_copy(b_hbm, bvec)
            bsplat16 = bvec[...] << 16

        ones = jnp.ones((L,), jnp.int32)

        for j in range(2):
            start_dma(base + j, j)

        wpl = w_dim // L
        wpl_shift = wpl.bit_length() - 1
        assert wpl == 1 << wpl_shift

        @pl.loop(0, n_chunks, step=2)
        def _outer(g):
            for j in range(2):
                gg = g + j
                pltpu.make_async_copy(
                    x_hbm.at[0, pl.ds(0, RB), :], bufs[j], sems[j]
                ).wait()

                @plsc.parallel_loop(0, nvec, unroll=16)
                def _inner(i):
                    r = lax.shift_right_logical(i, wpl_shift)
                    c = (i & (wpl - 1)) * L
                    bits = plsc.bitcast(bufs[j][r, pl.ds(c, L)], jnp.int32)
                    if pass2:
                        hi16 = bits & 0x7FFF0000
                        row = lax.shift_right_logical(bits, 7) & 0x1FF
                        col = bits & 127
                        plsc.addupdate_scatter(
                            hist, [row, col], ones, mask=(hi16 == bsplat16)
                        )
                    else:
                        row = lax.shift_right_logical(bits, 23) & 0xFF
                        col = lax.shift_right_logical(bits, 16) & 127
                        plsc.addupdate_scatter(hist, [row, col], ones)

                @pl.when(gg + 2 < n_chunks)
                def _next():
                    start_dma(base + gg + 2, j)

        pltpu.sync_copy(hist, out_hbm.at[wid])

    kern = pl.kernel(
        body,
        out_type=jax.ShapeDtypeStruct((NW,) + hist_shape, jnp.int32),
        mesh=mesh,
        scratch_types=scratch,
        compiler_params=pltpu.CompilerParams(
            needs_layout_passes=False, use_tc_tiling_on_sc=True
        ),
    )
    if pass2:
        return kern(x_bits, b16)
    return kern(x_bits)


def _cumulative(s):
    rows = s.shape[0]
    c = s
    sh = 1
    while sh < 128:
        c = c + jnp.concatenate(
            [jnp.zeros((rows, sh), jnp.int32), c[:, :-sh]], axis=1
        )
        sh *= 2
    rt = c[:, 127:128]
    e = rt
    sh = 1
    while sh < rows:
        e = e + jnp.concatenate(
            [jnp.zeros((sh, 1), jnp.int32), e[:-sh, :]], axis=0
        )
        sh *= 2
    return (e - rt) + c


def _tc_reduce1(h1, k):
    def body(h_ref, b_ref, r_ref):
        cum = _cumulative(jnp.sum(h_ref[...], axis=0))
        mask = cum <= k
        b = jnp.sum(mask.astype(jnp.int32))
        cum_before = jnp.max(jnp.where(mask, cum, 0))
        b_ref[...] = jnp.full((1, L), b, jnp.int32)
        r_ref[...] = jnp.full((1, 1), k - cum_before, jnp.int32)

    return pl.pallas_call(
        body,
        out_shape=(
            jax.ShapeDtypeStruct((1, L), jnp.int32),
            jax.ShapeDtypeStruct((1, 1), jnp.int32),
        ),
    )(h1)


def _tc_quantize(x3d, h2, b2d, r2d, g2d):
    a, m, w = x3d.shape
    bm = 512

    def body(b_ref, r_ref, g_ref, h_ref, x_ref, o_ref, sc_ref):
        @pl.when((pl.program_id(0) == 0) & (pl.program_id(1) == 0))
        def _scale():
            cum = _cumulative(jnp.sum(h_ref[...], axis=0))
            r = r_ref[0, 0]
            low = jnp.sum((cum <= r).astype(jnp.int32))
            qbits = (b_ref[0, 0] << 16) | low
            q = lax.bitcast_convert_type(qbits, jnp.float32)
            gc = jnp.clip(g_ref[0, 0], GAMMA_MIN, GAMMA_MAX)
            s = q * gc / Q_MAX
            sc_ref[0] = s
            sc_ref[1] = 1.0 / s

        s = sc_ref[0]
        si = sc_ref[1]
        q = jnp.clip(jnp.round(x_ref[...] * si), -Q_MAX, Q_MAX)
        o_ref[...] = q * s

    return pl.pallas_call(
        body,
        grid=(a, m // bm),
        in_specs=[
            pl.BlockSpec(memory_space=pltpu.SMEM),
            pl.BlockSpec(memory_space=pltpu.SMEM),
            pl.BlockSpec(memory_space=pltpu.SMEM),
            pl.BlockSpec(h2.shape, lambda i, j: (0, 0, 0)),
            pl.BlockSpec((1, bm, w), lambda i, j: (i, j, 0)),
        ],
        out_specs=pl.BlockSpec((1, bm, w), lambda i, j: (i, j, 0)),
        out_shape=jax.ShapeDtypeStruct((a, m, w), jnp.float32),
        scratch_shapes=[pltpu.SMEM((2,), jnp.float32)],
    )(b2d, r2d, g2d, h2, x3d)


def kernel(x, gamma):
    n = x.size
    k = round(QUANTILE * n)
    h1 = _sc_hist(x, None)
    b2d, r2d = _tc_reduce1(h1, k)
    h2 = _sc_hist(x, b2d.reshape(L))
    return _tc_quantize(x, h2, b2d, r2d, gamma.reshape(1, 1))
